# baseline (speedup 1.0000x reference)
import functools

import jax
import jax.numpy as jnp
import numpy as np
from jax import lax
from jax.experimental import pallas as pl
from jax.experimental.pallas import tpu as pltpu
from jax.experimental.pallas import tpu_sc as plsc


def _dist_argmin_body(z_ref, e_ref, iota_ref, idx_ref, acc_ref, *, num_e, scale):
    i = pl.program_id(0)
    z = z_ref[...]
    z2 = jnp.sum(z * z, axis=1, keepdims=True)
    zm2 = z * (-2.0)
    et = 256
    m_run = None
    idx_run = None
    for j in range(num_e // et):
        e = e_ref[pl.ds(j * et, et), :]
        dots = lax.dot_general(zm2, e, (((1,), (1,)), ((), ())),
                               preferred_element_type=jnp.float32)
        e2 = jnp.sum(e * e, axis=1)[None, :]
        dist = (z2 + dots) + e2
        m = jnp.min(dist, axis=1, keepdims=True)
        iota_j = iota_ref[:, pl.ds(j * et, et)]
        idx_f = jnp.min(jnp.where(dist == m, iota_j, float(num_e)), axis=1)
        if m_run is None:
            m_run, idx_run = m, idx_f
        else:
            better = m < m_run
            idx_run = jnp.where(better[:, 0], idx_f, idx_run)
            m_run = jnp.minimum(m, m_run)
    idx_ref[...] = idx_run.astype(jnp.int32)[None, None, :]

    @pl.when(i == 0)
    def _():
        acc_ref[...] = jnp.zeros_like(acc_ref)

    acc_ref[...] += jnp.sum(m_run) * scale


def _dist_argmin(z_flat, embedding, block_m, grid, block_off=0, interpret=False):
    n, d = z_flat.shape
    num_e = embedding.shape[0]
    scale = 1.0 / (n * d)
    return pl.pallas_call(
        functools.partial(_dist_argmin_body, num_e=num_e, scale=scale),
        grid=(grid,),
        in_specs=[
            pl.BlockSpec((block_m, d), lambda i: (i + block_off, 0)),
            pl.BlockSpec((num_e, d), lambda i: (0, 0)),
            pl.BlockSpec((1, num_e), lambda i: (0, 0)),
        ],
        out_specs=[
            pl.BlockSpec((1, 1, block_m), lambda i: (i, 0, 0)),
            pl.BlockSpec((1, 128), lambda i: (0, 0)),
        ],
        out_shape=[
            jax.ShapeDtypeStruct((grid, 1, block_m), jnp.int32),
            jax.ShapeDtypeStruct((1, 128), jnp.float32),
        ],
        interpret=interpret,
    )(z_flat, embedding, jnp.asarray(np.arange(num_e, dtype=np.float32)[None, :]))


def _make_sc_gather(num_e, d, rows, chunk_off, n_total, as_output):
    info = plsc.get_sparse_core_info()
    nw = info.num_cores * info.num_subcores
    b_per_w = rows // nw
    mesh = plsc.VectorSubcoreMesh(core_axis_name="c", subcore_axis_name="s")

    @functools.partial(
        pl.kernel,
        mesh=mesh,
        out_type=(jax.ShapeDtypeStruct((n_total, d), jnp.float32)
                  if as_output else ()),
        scratch_types=[
            pltpu.VMEM((b_per_w,), jnp.int32),
            pltpu.VMEM((b_per_w, d), jnp.float32),
            pltpu.SemaphoreType.DMA,
        ],
    )
    def gather_k(table_hbm, idx_hbm, out_hbm, idx_v, rows_v, gsem):
        wid = lax.axis_index("s") * info.num_cores + lax.axis_index("c")
        base = wid * b_per_w
        pltpu.sync_copy(idx_hbm.at[pl.ds(base, b_per_w)], idx_v)
        pltpu.async_copy(table_hbm.at[idx_v], rows_v, gsem).wait()
        pltpu.sync_copy(rows_v, out_hbm.at[pl.ds(chunk_off + base, b_per_w)])

    return gather_k


def kernel(z, embedding):
    b, t, d = z.shape
    num_e = embedding.shape[0]
    n = b * t
    nchunks = 2
    blocks = b // nchunks
    rows = n // nchunks
    z_flat = z.reshape(n, d)
    idx_parts, accs = [], []
    zq_ref = None
    for c in range(nchunks):
        idx2d, acc = _dist_argmin(z_flat, embedding, block_m=t,
                                  grid=blocks, block_off=c * blocks)
        gather = _make_sc_gather(num_e, d, rows, c * rows, n, as_output=(c == 0))
        if c == 0:
            zq_ref = jax.new_ref(gather(embedding, idx2d.reshape(rows)))
        else:
            gather(embedding, idx2d.reshape(rows), zq_ref)
        idx_parts.append(idx2d)
        accs.append(acc)
    z_q = zq_ref[...].reshape(b, t, d)
    indices = jnp.concatenate(idx_parts, axis=0).reshape(b, t)
    loss = accs[0][0, 0]
    for a in accs[1:]:
        loss = loss + a[0, 0]
    return (z_q, indices, loss)


# speedup vs baseline: 1.1971x; 1.1971x over previous
import functools

import jax
import jax.numpy as jnp
import numpy as np
from jax import lax
from jax.experimental import pallas as pl
from jax.experimental.pallas import tpu as pltpu
from jax.experimental.pallas import tpu_sc as plsc


def _dist_argmin_body(z_ref, e_ref, iota_ref, idx_ref, acc_ref, *, num_e, scale):
    i = pl.program_id(0)
    z = z_ref[...]
    z2 = jnp.sum(z * z, axis=1, keepdims=True)
    zm2 = z * (-2.0)
    et = 256
    m_run = None
    idx_run = None
    for j in range(num_e // et):
        e = e_ref[pl.ds(j * et, et), :]
        dots = lax.dot_general(zm2, e, (((1,), (1,)), ((), ())),
                               preferred_element_type=jnp.float32)
        e2 = jnp.sum(e * e, axis=1)[None, :]
        dist = (z2 + dots) + e2
        m = jnp.min(dist, axis=1, keepdims=True)
        iota_j = iota_ref[:, pl.ds(j * et, et)]
        idx_f = jnp.min(jnp.where(dist == m, iota_j, float(num_e)), axis=1)
        if m_run is None:
            m_run, idx_run = m, idx_f
        else:
            better = m < m_run
            idx_run = jnp.where(better[:, 0], idx_f, idx_run)
            m_run = jnp.minimum(m, m_run)
    idx_ref[...] = idx_run.astype(jnp.int32)[None, None, :]

    @pl.when(i == 0)
    def _():
        acc_ref[...] = jnp.zeros_like(acc_ref)

    acc_ref[...] += jnp.sum(m_run) * scale


def _dist_argmin(z_flat, embedding, block_m, grid, block_off=0, interpret=False):
    n, d = z_flat.shape
    num_e = embedding.shape[0]
    scale = 1.0 / (n * d)
    return pl.pallas_call(
        functools.partial(_dist_argmin_body, num_e=num_e, scale=scale),
        grid=(grid,),
        in_specs=[
            pl.BlockSpec((block_m, d), lambda i: (i + block_off, 0)),
            pl.BlockSpec((num_e, d), lambda i: (0, 0)),
            pl.BlockSpec((1, num_e), lambda i: (0, 0)),
        ],
        out_specs=[
            pl.BlockSpec((1, 1, block_m), lambda i: (i, 0, 0)),
            pl.BlockSpec((1, 128), lambda i: (0, 0)),
        ],
        out_shape=[
            jax.ShapeDtypeStruct((grid, 1, block_m), jnp.int32),
            jax.ShapeDtypeStruct((1, 128), jnp.float32),
        ],
        interpret=interpret,
    )(z_flat, embedding, jnp.asarray(np.arange(num_e, dtype=np.float32)[None, :]))


def _make_sc_gather(num_e, d, rows, chunk_off, n_total, as_output):
    info = plsc.get_sparse_core_info()
    nw = info.num_cores * info.num_subcores
    b_per_w = rows // nw
    mesh = plsc.VectorSubcoreMesh(core_axis_name="c", subcore_axis_name="s")

    @functools.partial(
        pl.kernel,
        mesh=mesh,
        out_type=(jax.ShapeDtypeStruct((n_total, d), jnp.float32)
                  if as_output else ()),
        scratch_types=[
            pltpu.VMEM((b_per_w,), jnp.int32),
            pltpu.VMEM((b_per_w, d), jnp.float32),
            pltpu.SemaphoreType.DMA,
        ],
    )
    def gather_k(table_hbm, idx_hbm, out_hbm, idx_v, rows_v, gsem):
        wid = lax.axis_index("s") * info.num_cores + lax.axis_index("c")
        base = wid * b_per_w
        pltpu.sync_copy(idx_hbm.at[pl.ds(base, b_per_w)], idx_v)
        pltpu.async_copy(table_hbm.at[idx_v], rows_v, gsem).wait()
        pltpu.sync_copy(rows_v, out_hbm.at[pl.ds(chunk_off + base, b_per_w)])

    return gather_k


def _onehot_gather_body(idx_ref, e_ref, buf_ref, out_ref):
    del buf_ref
    idx = idx_ref[0, 0, :]
    iota = lax.broadcasted_iota(jnp.int32, (idx.shape[0], e_ref.shape[0]), 1)
    onehot = jnp.where(iota == idx[:, None], 1.0, 0.0)
    out_ref[...] = lax.dot_general(onehot, e_ref[...], (((1,), (0,)), ((), ())),
                                   preferred_element_type=jnp.float32)


def _onehot_gather(idx3d, embedding, buf, block_off):
    grid, _, block_m = idx3d.shape
    num_e, d = embedding.shape
    n = buf.shape[0]
    return pl.pallas_call(
        _onehot_gather_body,
        grid=(grid,),
        in_specs=[
            pl.BlockSpec((1, 1, block_m), lambda i: (i, 0, 0)),
            pl.BlockSpec((num_e, d), lambda i: (0, 0)),
            pl.BlockSpec(memory_space=pl.ANY),
        ],
        out_specs=pl.BlockSpec((block_m, d), lambda i: (i + block_off, 0)),
        out_shape=jax.ShapeDtypeStruct((n, d), jnp.float32),
        input_output_aliases={2: 0},
    )(idx3d, embedding, buf)


def kernel(z, embedding):
    b, t, d = z.shape
    num_e = embedding.shape[0]
    n = b * t
    nchunks = 2
    blocks = b // nchunks
    rows = n // nchunks
    z_flat = z.reshape(n, d)
    idx1, acc1 = _dist_argmin(z_flat, embedding, block_m=t, grid=blocks)
    zq1 = _make_sc_gather(num_e, d, rows, 0, n, as_output=True)(
        embedding, idx1.reshape(rows))
    idx2, acc2 = _dist_argmin(z_flat, embedding, block_m=t,
                              grid=blocks, block_off=blocks)
    zq = _onehot_gather(idx2, embedding, zq1, blocks)
    z_q = zq.reshape(b, t, d)
    indices = jnp.concatenate([idx1, idx2], axis=0).reshape(b, t)
    loss = acc1[0, 0] + acc2[0, 0]
    return (z_q, indices, loss)
